# baseline (speedup 1.0000x reference)
import functools

import jax
import jax.numpy as jnp
import numpy as np
from jax import lax
from jax.experimental import pallas as pl
from jax.experimental.pallas import tpu as pltpu
from jax.experimental.pallas import tpu_sc as plsc

B, LQ, D = 4, 1024, 256
H, P = 8, 4
HH, WW = 100, 100
LIN = HH * WW
DH = D // H
NPTS = P * 4
NOUT = B * LQ * H

_S = np.zeros((32, 32), np.float32)
for _i in range(32):
    for _j in range(32):
        if _i // 4 == _j // 4:
            _S[_i, _j] = 1.0
_A1 = np.zeros((32, 128), np.float32)
_MYX = np.zeros((128, 256), np.float32)
for _h in range(H):
    for _p in range(P):
        for _c in range(4):
            _l = _h * 16 + _p * 4 + _c
            _A1[_h * 4 + _p, _l] = 1.0
            _MYX[(_c >> 1) * 64 + _h * 8 + _p * 2 + 1, _l] = 1.0
            _MYX[(_c & 1) * 64 + _h * 8 + _p * 2 + 0, 128 + _l] = 1.0

_VBLK = 2000


def _valproj_body(src_ref, w_ref, b_ref, out_ref):
    r = (jnp.dot(src_ref[:], w_ref[:], preferred_element_type=jnp.float32)
         + b_ref[:])
    lo = jax.lax.bitcast_convert_type(
        r[:, :128].astype(jnp.bfloat16), jnp.uint16).astype(jnp.uint32)
    hi = jax.lax.bitcast_convert_type(
        r[:, 128:].astype(jnp.bfloat16), jnp.uint16).astype(jnp.uint32)
    out_ref[:] = jax.lax.bitcast_convert_type((hi << 16) | lo, jnp.float32)


def _valproj(src_f, W_val, b_val, half):
    rows = 2 * LIN
    grid = rows // _VBLK
    off = half * grid
    return pl.pallas_call(
        _valproj_body,
        grid=(grid,),
        in_specs=[
            pl.BlockSpec((_VBLK, D), lambda i: (i + off, 0)),
            pl.BlockSpec((D, D), lambda i: (0, 0)),
            pl.BlockSpec((1, D), lambda i: (0, 0)),
        ],
        out_specs=pl.BlockSpec((_VBLK, 128), lambda i: (i, 0)),
        out_shape=jax.ShapeDtypeStruct((rows, 128), jnp.float32),
    )(src_f, W_val, b_val)


_QBLK = 512


def _prep_body(tgt_ref, rp_ref, woff_ref, boff_ref, wattn_ref, battn_ref,
               s_ref, a1_ref, myx_ref, w_out_ref, idx_out_ref):
    i = pl.program_id(0)
    b_f = (i // (LQ // _QBLK)).astype(jnp.float32)
    t = tgt_ref[:]
    off = jnp.dot(t, woff_ref[:], preferred_element_type=jnp.float32) + boff_ref[:]
    a = jnp.dot(t, wattn_ref[:], preferred_element_type=jnp.float32) + battn_ref[:]
    a = a - jnp.max(a, axis=1, keepdims=True)
    ea = jnp.exp(a)
    gs = jnp.dot(ea, s_ref[:], preferred_element_type=jnp.float32)
    aw = ea / gs

    lane64 = lax.broadcasted_iota(jnp.int32, (_QBLK, 64), 1)
    is_x = (lane64 % 2) == 0
    rp = rp_ref[:]
    refc = jnp.where(is_x, rp[:, 0:1], rp[:, 1:2])
    loc = (refc + off / 100.0) * 100.0 - 0.5
    fl = jnp.floor(loc)
    w1 = loc - fl
    w0 = 1.0 - w1
    v0 = ((fl >= 0.0) & (fl <= 99.0)).astype(jnp.float32)
    v1 = ((fl >= -1.0) & (fl <= 98.0)).astype(jnp.float32)
    c0 = jnp.clip(fl, 0.0, 99.0)
    c1 = jnp.clip(fl + 1.0, 0.0, 99.0)

    MYX = myx_ref[:]
    wyx = jnp.dot(jnp.concatenate([w0, w1], axis=1), MYX,
                  preferred_element_type=jnp.float32)
    vyx = jnp.dot(jnp.concatenate([v0, v1], axis=1), MYX,
                  preferred_element_type=jnp.float32)
    cyx = jnp.dot(jnp.concatenate([c0, c1], axis=1), MYX,
                  preferred_element_type=jnp.float32)
    aw128 = jnp.dot(aw, a1_ref[:], preferred_element_type=jnp.float32)

    h128 = (lax.broadcasted_iota(jnp.int32, (_QBLK, 128), 1) // 16).astype(jnp.float32)
    base = jnp.mod(b_f, 2.0) * float(LIN * H)
    idxf = (cyx[:, :128] * 100.0 + cyx[:, 128:]) * 8.0 + h128 + base
    w_out_ref[:] = (aw128 * wyx[:, :128] * wyx[:, 128:]
                    * vyx[:, :128] * vyx[:, 128:])
    idx_out_ref[:] = idxf.astype(jnp.int32)


def _prep(tgt_f, rp_f, W_off, b_off, W_attn, b_attn):
    grid = (B * LQ) // _QBLK
    return pl.pallas_call(
        _prep_body,
        grid=(grid,),
        in_specs=[
            pl.BlockSpec((_QBLK, D), lambda i: (i, 0)),
            pl.BlockSpec((_QBLK, 2), lambda i: (i, 0)),
            pl.BlockSpec((D, 64), lambda i: (0, 0)),
            pl.BlockSpec((1, 64), lambda i: (0, 0)),
            pl.BlockSpec((D, 32), lambda i: (0, 0)),
            pl.BlockSpec((1, 32), lambda i: (0, 0)),
            pl.BlockSpec((32, 32), lambda i: (0, 0)),
            pl.BlockSpec((32, 128), lambda i: (0, 0)),
            pl.BlockSpec((128, 256), lambda i: (0, 0)),
        ],
        out_specs=[
            pl.BlockSpec((_QBLK, 128), lambda i: (i, 0)),
            pl.BlockSpec((_QBLK, 128), lambda i: (i, 0)),
        ],
        out_shape=[
            jax.ShapeDtypeStruct((B * LQ, 128), jnp.float32),
            jax.ShapeDtypeStruct((B * LQ, 128), jnp.int32),
        ],
    )(tgt_f, rp_f, W_off, b_off, W_attn, b_attn, _S, _A1, _MYX)


_NW = 32
_NOUT_H = NOUT // 2
_RPW = _NOUT_H // _NW
_CH = 128
_NCHUNK = _RPW // _CH
_NGATH = (_CH * NPTS) // 128

_NBUF = 3


def _sc_gather_impl(half, table_hbm, idx_hbm, w_hbm, out_hbm, *scr):
    wid = lax.axis_index("s") * 2 + lax.axis_index("c")
    row0 = half * _NOUT_H
    bufs = tuple(dict(idx=scr[b], w=scr[_NBUF + b], rows=scr[2 * _NBUF + b],
                      out=scr[3 * _NBUF + b], isem=scr[4 * _NBUF + b],
                      gsem=scr[5 * _NBUF + b], osem=scr[6 * _NBUF + b])
                 for b in range(_NBUF))

    def start_idxw(c, buf):
        base = wid * _RPW + c * _CH
        return [pltpu.async_copy(
                    idx_hbm.at[pl.ds((row0 + base) * NPTS // 128, _NGATH)],
                    buf["idx"], buf["isem"]),
                pltpu.async_copy(
                    w_hbm.at[pl.ds((row0 + base) * NPTS, _CH * NPTS)],
                    buf["w"], buf["isem"])]

    def fire(buf):
        return [pltpu.async_copy(table_hbm.at[buf["idx"].at[j]],
                                 buf["rows"].at[pl.ds(j * 128, 128)],
                                 buf["gsem"])
                for j in range(_NGATH)]

    def compute(c, buf, gathers):
        w_v, rows_v, out_v = buf["w"], buf["rows"], buf["out"]
        base = wid * _RPW + c * _CH
        for cp in gathers:
            cp.wait()

        @plsc.parallel_loop(0, _CH, 1, unroll=2)
        def j_body(j):
            wv = w_v[pl.ds(j * NPTS, 16)]
            acc0 = jnp.zeros((16,), jnp.float32)
            acc1 = jnp.zeros((16,), jnp.float32)
            for i in range(NPTS):
                k = j * NPTS + i
                wrd = rows_v[k, pl.ds(0, 16)]
                u = plsc.bitcast(wrd, jnp.uint32)
                g0 = plsc.bitcast(u << 16, jnp.float32)
                g1 = plsc.bitcast(u & jnp.uint32(0xFFFF0000), jnp.float32)
                ws = lax.gather(
                    wv, jnp.full((16, 1), i, jnp.int32),
                    lax.GatherDimensionNumbers(
                        offset_dims=(), collapsed_slice_dims=(0,),
                        start_index_map=(0,)),
                    (1,), mode=lax.GatherScatterMode.PROMISE_IN_BOUNDS)
                acc0 = acc0 + g0 * ws
                acc1 = acc1 + g1 * ws
            out_v[pl.ds(j * DH, 16)] = acc0
            out_v[pl.ds(j * DH + 16, 16)] = acc1
        return pltpu.async_copy(out_v, out_hbm.at[pl.ds(base * DH, _CH * DH)],
                                buf["osem"])

    iw = [None] * _NCHUNK
    gth = [None] * _NCHUNK
    ocp = [None] * _NBUF
    iw[0] = start_idxw(0, bufs[0])
    if _NCHUNK > 1:
        iw[1] = start_idxw(1, bufs[1])
    for cp in iw[0]:
        cp.wait()
    gth[0] = fire(bufs[0])
    for c in range(_NCHUNK):
        if c + 2 < _NCHUNK:
            iw[c + 2] = start_idxw(c + 2, bufs[(c + 2) % _NBUF])
        if c + 1 < _NCHUNK:
            for cp in iw[c + 1]:
                cp.wait()
            gth[c + 1] = fire(bufs[(c + 1) % _NBUF])
        if ocp[c % _NBUF] is not None:
            ocp[c % _NBUF].wait()
        ocp[c % _NBUF] = compute(c, bufs[c % _NBUF], gth[c])
    for oc in ocp:
        if oc is not None:
            oc.wait()


@functools.cache
def _sc_gather_kernel(half):
    mesh = plsc.VectorSubcoreMesh(
        core_axis_name="c", subcore_axis_name="s", num_cores=2, num_subcores=16)
    scratch = ([pltpu.VMEM((_NGATH, 128), jnp.int32)] * _NBUF
               + [pltpu.VMEM((_CH * NPTS,), jnp.float32)] * _NBUF
               + [pltpu.VMEM((_CH * NPTS, 16), jnp.float32)] * _NBUF
               + [pltpu.VMEM((_CH * DH,), jnp.float32)] * _NBUF
               + [pltpu.SemaphoreType.DMA] * (3 * _NBUF))
    return pl.kernel(
        functools.partial(_sc_gather_impl, half),
        out_type=jax.ShapeDtypeStruct((_NOUT_H * DH,), jnp.float32),
        mesh=mesh,
        scratch_types=scratch,
        compiler_params=pltpu.CompilerParams(use_tc_tiling_on_sc=False,
                                             needs_layout_passes=False),
    )


_FBLK = 512


def _final_body(attn_ref, tgt_ref, w_ref, b_ref, g_ref, bn_ref, out_ref):
    y = (jnp.dot(attn_ref[:], w_ref[:], preferred_element_type=jnp.float32)
         + b_ref[:] + tgt_ref[:])
    m = jnp.mean(y, axis=1, keepdims=True)
    yc = y - m
    v = jnp.mean(yc * yc, axis=1, keepdims=True)
    out_ref[:] = yc / jnp.sqrt(v + 1e-5) * g_ref[:] + bn_ref[:]


def _final(attn_f, tgt_f, W_out, b_out, g1, b1n):
    grid = (B * LQ) // _FBLK
    return pl.pallas_call(
        _final_body,
        grid=(grid,),
        in_specs=[
            pl.BlockSpec((_FBLK, D), lambda i: (i, 0)),
            pl.BlockSpec((_FBLK, D), lambda i: (i, 0)),
            pl.BlockSpec((D, D), lambda i: (0, 0)),
            pl.BlockSpec((1, D), lambda i: (0, 0)),
            pl.BlockSpec((1, D), lambda i: (0, 0)),
            pl.BlockSpec((1, D), lambda i: (0, 0)),
        ],
        out_specs=pl.BlockSpec((_FBLK, D), lambda i: (i, 0)),
        out_shape=jax.ShapeDtypeStruct((B * LQ, D), jnp.float32),
    )(attn_f, tgt_f, W_out, b_out, g1, b1n)


def kernel(tgt, src, reference_points, spatial_shapes, level_start_index,
           W_off, b_off, W_attn, b_attn, W_val, b_val, W_out, b_out,
           g1, b1n, W1, bl1, W2, bl2, g2, b2n):
    tgt_f = tgt.reshape(B * LQ, D)
    src_f = src.reshape(B * LIN, D)
    rp_f = reference_points.reshape(B * LQ, 2)

    Wv3 = W_val.reshape(D, H, DH)
    Wp = jnp.concatenate([Wv3[:, :, :16].reshape(D, H * 16),
                          Wv3[:, :, 16:].reshape(D, H * 16)], axis=1)
    bv = b_val.reshape(H, DH)
    bp = jnp.concatenate([bv[:, :16].reshape(-1), bv[:, 16:].reshape(-1)])
    value0 = _valproj(src_f, Wp, bp.reshape(1, D), 0)
    value1 = _valproj(src_f, Wp, bp.reshape(1, D), 1)
    w128, idx128 = _prep(tgt_f, rp_f, W_off, b_off.reshape(1, 64),
                         W_attn, b_attn.reshape(1, 32))

    w_flat = w128.reshape(-1)
    attn0 = _sc_gather_kernel(0)(value0.reshape(2 * LIN * H, 16), idx128, w_flat)
    attn1 = _sc_gather_kernel(1)(value1.reshape(2 * LIN * H, 16), idx128, w_flat)
    attn_flat = jnp.concatenate([attn0, attn1])

    out = _final(attn_flat.reshape(B * LQ, D), tgt_f,
                 W_out, b_out.reshape(1, D), g1.reshape(1, D), b1n.reshape(1, D))
    return out.reshape(B, LQ, D)


# speedup vs baseline: 1.0157x; 1.0157x over previous
import functools

import jax
import jax.numpy as jnp
import numpy as np
from jax import lax
from jax.experimental import pallas as pl
from jax.experimental.pallas import tpu as pltpu
from jax.experimental.pallas import tpu_sc as plsc

B, LQ, D = 4, 1024, 256
H, P = 8, 4
HH, WW = 100, 100
LIN = HH * WW
DH = D // H
NPTS = P * 4
NOUT = B * LQ * H

_S = np.zeros((32, 32), np.float32)
for _i in range(32):
    for _j in range(32):
        if _i // 4 == _j // 4:
            _S[_i, _j] = 1.0
_A1 = np.zeros((32, 128), np.float32)
_MYX = np.zeros((128, 256), np.float32)
for _h in range(H):
    for _p in range(P):
        for _c in range(4):
            _l = _h * 16 + _p * 4 + _c
            _A1[_h * 4 + _p, _l] = 1.0
            _MYX[(_c >> 1) * 64 + _h * 8 + _p * 2 + 1, _l] = 1.0
            _MYX[(_c & 1) * 64 + _h * 8 + _p * 2 + 0, 128 + _l] = 1.0

_VBLK = 2000


def _valproj_body(src_ref, w_ref, b_ref, out_ref):
    r = (jnp.dot(src_ref[:], w_ref[:], preferred_element_type=jnp.float32)
         + b_ref[:])
    lo = jax.lax.bitcast_convert_type(
        r[:, :128].astype(jnp.bfloat16), jnp.uint16).astype(jnp.uint32)
    hi = jax.lax.bitcast_convert_type(
        r[:, 128:].astype(jnp.bfloat16), jnp.uint16).astype(jnp.uint32)
    out_ref[:] = jax.lax.bitcast_convert_type((hi << 16) | lo, jnp.float32)


def _valproj(src_f, W_val, b_val):
    rows = B * LIN
    grid = rows // _VBLK
    return pl.pallas_call(
        _valproj_body,
        grid=(grid,),
        in_specs=[
            pl.BlockSpec((_VBLK, D), lambda i: (i, 0)),
            pl.BlockSpec((D, D), lambda i: (0, 0)),
            pl.BlockSpec((1, D), lambda i: (0, 0)),
        ],
        out_specs=pl.BlockSpec((_VBLK, 128), lambda i: (i, 0)),
        out_shape=jax.ShapeDtypeStruct((rows, 128), jnp.float32),
    )(src_f, W_val, b_val)


_QBLK = 512


def _prep_body(tgt_ref, rp_ref, woff_ref, boff_ref, wattn_ref, battn_ref,
               s_ref, a1_ref, myx_ref, w_out_ref, idx_out_ref):
    i = pl.program_id(0)
    b_f = (i // (LQ // _QBLK)).astype(jnp.float32)
    t = tgt_ref[:]
    off = jnp.dot(t, woff_ref[:], preferred_element_type=jnp.float32) + boff_ref[:]
    a = jnp.dot(t, wattn_ref[:], preferred_element_type=jnp.float32) + battn_ref[:]
    a = a - jnp.max(a, axis=1, keepdims=True)
    ea = jnp.exp(a)
    gs = jnp.dot(ea, s_ref[:], preferred_element_type=jnp.float32)
    aw = ea / gs

    lane64 = lax.broadcasted_iota(jnp.int32, (_QBLK, 64), 1)
    is_x = (lane64 % 2) == 0
    rp = rp_ref[:]
    refc = jnp.where(is_x, rp[:, 0:1], rp[:, 1:2])
    loc = (refc + off / 100.0) * 100.0 - 0.5
    fl = jnp.floor(loc)
    w1 = loc - fl
    w0 = 1.0 - w1
    v0 = ((fl >= 0.0) & (fl <= 99.0)).astype(jnp.float32)
    v1 = ((fl >= -1.0) & (fl <= 98.0)).astype(jnp.float32)
    c0 = jnp.clip(fl, 0.0, 99.0)
    c1 = jnp.clip(fl + 1.0, 0.0, 99.0)

    MYX = myx_ref[:]
    wyx = jnp.dot(jnp.concatenate([w0, w1], axis=1), MYX,
                  preferred_element_type=jnp.float32)
    vyx = jnp.dot(jnp.concatenate([v0, v1], axis=1), MYX,
                  preferred_element_type=jnp.float32)
    cyx = jnp.dot(jnp.concatenate([c0, c1], axis=1), MYX,
                  preferred_element_type=jnp.float32)
    aw128 = jnp.dot(aw, a1_ref[:], preferred_element_type=jnp.float32)

    h128 = (lax.broadcasted_iota(jnp.int32, (_QBLK, 128), 1) // 16).astype(jnp.float32)
    base = b_f * float(LIN * H)
    idxf = (cyx[:, :128] * 100.0 + cyx[:, 128:]) * 8.0 + h128 + base
    w_out_ref[:] = (aw128 * wyx[:, :128] * wyx[:, 128:]
                    * vyx[:, :128] * vyx[:, 128:])
    idx_out_ref[:] = idxf.astype(jnp.int32)


def _prep(tgt_f, rp_f, W_off, b_off, W_attn, b_attn):
    grid = (B * LQ) // _QBLK
    return pl.pallas_call(
        _prep_body,
        grid=(grid,),
        in_specs=[
            pl.BlockSpec((_QBLK, D), lambda i: (i, 0)),
            pl.BlockSpec((_QBLK, 2), lambda i: (i, 0)),
            pl.BlockSpec((D, 64), lambda i: (0, 0)),
            pl.BlockSpec((1, 64), lambda i: (0, 0)),
            pl.BlockSpec((D, 32), lambda i: (0, 0)),
            pl.BlockSpec((1, 32), lambda i: (0, 0)),
            pl.BlockSpec((32, 32), lambda i: (0, 0)),
            pl.BlockSpec((32, 128), lambda i: (0, 0)),
            pl.BlockSpec((128, 256), lambda i: (0, 0)),
        ],
        out_specs=[
            pl.BlockSpec((_QBLK, 128), lambda i: (i, 0)),
            pl.BlockSpec((_QBLK, 128), lambda i: (i, 0)),
        ],
        out_shape=[
            jax.ShapeDtypeStruct((B * LQ, 128), jnp.float32),
            jax.ShapeDtypeStruct((B * LQ, 128), jnp.int32),
        ],
    )(tgt_f, rp_f, W_off, b_off, W_attn, b_attn, _S, _A1, _MYX)


_NW = 32
_RPW = NOUT // _NW
_CH = 128
_NCHUNK = _RPW // _CH
_NGATH = (_CH * NPTS) // 128

_NBUF = 3


def _sc_gather_impl(table_hbm, idx_hbm, w_hbm, out_hbm, *scr):
    wid = lax.axis_index("s") * 2 + lax.axis_index("c")
    bufs = tuple(dict(idx=scr[b], w=scr[_NBUF + b], rows=scr[2 * _NBUF + b],
                      out=scr[3 * _NBUF + b], isem=scr[4 * _NBUF + b],
                      gsem=scr[5 * _NBUF + b], osem=scr[6 * _NBUF + b])
                 for b in range(_NBUF))

    def start_idxw(c, buf):
        base = wid * _RPW + c * _CH
        return [pltpu.async_copy(
                    idx_hbm.at[pl.ds(base * NPTS // 128, _NGATH)],
                    buf["idx"], buf["isem"]),
                pltpu.async_copy(
                    w_hbm.at[pl.ds(base * NPTS, _CH * NPTS)],
                    buf["w"], buf["isem"])]

    def fire(buf):
        return [pltpu.async_copy(table_hbm.at[buf["idx"].at[j]],
                                 buf["rows"].at[pl.ds(j * 128, 128)],
                                 buf["gsem"])
                for j in range(_NGATH)]

    def compute(c, buf, gathers):
        w_v, rows_v, out_v = buf["w"], buf["rows"], buf["out"]
        base = wid * _RPW + c * _CH
        for cp in gathers:
            cp.wait()

        @plsc.parallel_loop(0, _CH, 1, unroll=2)
        def j_body(j):
            wv = w_v[pl.ds(j * NPTS, 16)]
            acc0 = jnp.zeros((16,), jnp.float32)
            acc1 = jnp.zeros((16,), jnp.float32)
            for i in range(NPTS):
                k = j * NPTS + i
                wrd = rows_v[k, pl.ds(0, 16)]
                u = plsc.bitcast(wrd, jnp.uint32)
                g0 = plsc.bitcast(u << 16, jnp.float32)
                g1 = plsc.bitcast(u & jnp.uint32(0xFFFF0000), jnp.float32)
                ws = lax.gather(
                    wv, jnp.full((16, 1), i, jnp.int32),
                    lax.GatherDimensionNumbers(
                        offset_dims=(), collapsed_slice_dims=(0,),
                        start_index_map=(0,)),
                    (1,), mode=lax.GatherScatterMode.PROMISE_IN_BOUNDS)
                acc0 = acc0 + g0 * ws
                acc1 = acc1 + g1 * ws
            out_v[pl.ds(j * DH, 16)] = acc0
            out_v[pl.ds(j * DH + 16, 16)] = acc1
        return pltpu.async_copy(out_v, out_hbm.at[pl.ds(base * DH, _CH * DH)],
                                buf["osem"])

    iw = [None] * _NCHUNK
    gth = [None] * _NCHUNK
    ocp = [None] * _NBUF
    iw[0] = start_idxw(0, bufs[0])
    if _NCHUNK > 1:
        iw[1] = start_idxw(1, bufs[1])
    for cp in iw[0]:
        cp.wait()
    gth[0] = fire(bufs[0])
    for c in range(_NCHUNK):
        if c + 2 < _NCHUNK:
            iw[c + 2] = start_idxw(c + 2, bufs[(c + 2) % _NBUF])
        if c + 1 < _NCHUNK:
            for cp in iw[c + 1]:
                cp.wait()
            gth[c + 1] = fire(bufs[(c + 1) % _NBUF])
        if ocp[c % _NBUF] is not None:
            ocp[c % _NBUF].wait()
        ocp[c % _NBUF] = compute(c, bufs[c % _NBUF], gth[c])
    for oc in ocp:
        if oc is not None:
            oc.wait()


@functools.cache
def _sc_gather_kernel():
    mesh = plsc.VectorSubcoreMesh(
        core_axis_name="c", subcore_axis_name="s", num_cores=2, num_subcores=16)
    scratch = ([pltpu.VMEM((_NGATH, 128), jnp.int32)] * _NBUF
               + [pltpu.VMEM((_CH * NPTS,), jnp.float32)] * _NBUF
               + [pltpu.VMEM((_CH * NPTS, 16), jnp.float32)] * _NBUF
               + [pltpu.VMEM((_CH * DH,), jnp.float32)] * _NBUF
               + [pltpu.SemaphoreType.DMA] * (3 * _NBUF))
    return pl.kernel(
        _sc_gather_impl,
        out_type=jax.ShapeDtypeStruct((NOUT * DH,), jnp.float32),
        mesh=mesh,
        scratch_types=scratch,
        compiler_params=pltpu.CompilerParams(use_tc_tiling_on_sc=False,
                                             needs_layout_passes=False),
    )


_FBLK = 512


def _final_body(attn_ref, tgt_ref, w_ref, b_ref, g_ref, bn_ref, out_ref):
    y = (jnp.dot(attn_ref[:], w_ref[:], preferred_element_type=jnp.float32)
         + b_ref[:] + tgt_ref[:])
    m = jnp.mean(y, axis=1, keepdims=True)
    yc = y - m
    v = jnp.mean(yc * yc, axis=1, keepdims=True)
    out_ref[:] = yc / jnp.sqrt(v + 1e-5) * g_ref[:] + bn_ref[:]


def _final(attn_f, tgt_f, W_out, b_out, g1, b1n):
    grid = (B * LQ) // _FBLK
    return pl.pallas_call(
        _final_body,
        grid=(grid,),
        in_specs=[
            pl.BlockSpec((_FBLK, D), lambda i: (i, 0)),
            pl.BlockSpec((_FBLK, D), lambda i: (i, 0)),
            pl.BlockSpec((D, D), lambda i: (0, 0)),
            pl.BlockSpec((1, D), lambda i: (0, 0)),
            pl.BlockSpec((1, D), lambda i: (0, 0)),
            pl.BlockSpec((1, D), lambda i: (0, 0)),
        ],
        out_specs=pl.BlockSpec((_FBLK, D), lambda i: (i, 0)),
        out_shape=jax.ShapeDtypeStruct((B * LQ, D), jnp.float32),
    )(attn_f, tgt_f, W_out, b_out, g1, b1n)


def kernel(tgt, src, reference_points, spatial_shapes, level_start_index,
           W_off, b_off, W_attn, b_attn, W_val, b_val, W_out, b_out,
           g1, b1n, W1, bl1, W2, bl2, g2, b2n):
    tgt_f = tgt.reshape(B * LQ, D)
    src_f = src.reshape(B * LIN, D)
    rp_f = reference_points.reshape(B * LQ, 2)

    Wv3 = W_val.reshape(D, H, DH)
    Wp = jnp.concatenate([Wv3[:, :, :16].reshape(D, H * 16),
                          Wv3[:, :, 16:].reshape(D, H * 16)], axis=1)
    bv = b_val.reshape(H, DH)
    bp = jnp.concatenate([bv[:, :16].reshape(-1), bv[:, 16:].reshape(-1)])
    value = _valproj(src_f, Wp, bp.reshape(1, D))
    w128, idx128 = _prep(tgt_f, rp_f, W_off, b_off.reshape(1, 64),
                         W_attn, b_attn.reshape(1, 32))

    attn_flat = _sc_gather_kernel()(value.reshape(B * LIN * H, 16), idx128,
                                    w128.reshape(-1))

    out = _final(attn_flat.reshape(B * LQ, D), tgt_f,
                 W_out, b_out.reshape(1, D), g1.reshape(1, D), b1n.reshape(1, D))
    return out.reshape(B, LQ, D)


# speedup vs baseline: 1.0852x; 1.0684x over previous
import functools

import jax
import jax.numpy as jnp
import numpy as np
from jax import lax
from jax.experimental import pallas as pl
from jax.experimental.pallas import tpu as pltpu
from jax.experimental.pallas import tpu_sc as plsc

B, LQ, D = 4, 1024, 256
H, P = 8, 4
HH, WW = 100, 100
LIN = HH * WW
DH = D // H
NPTS = P * 4
NOUT = B * LQ * H

_S = np.zeros((32, 32), np.float32)
for _i in range(32):
    for _j in range(32):
        if _i // 4 == _j // 4:
            _S[_i, _j] = 1.0
_A1 = np.zeros((32, 128), np.float32)
_MYX = np.zeros((128, 256), np.float32)
for _h in range(H):
    for _p in range(P):
        for _c in range(4):
            _l = _h * 16 + _p * 4 + _c
            _A1[_h * 4 + _p, _l] = 1.0
            _MYX[(_c >> 1) * 64 + _h * 8 + _p * 2 + 1, _l] = 1.0
            _MYX[(_c & 1) * 64 + _h * 8 + _p * 2 + 0, 128 + _l] = 1.0

_VBLK = 4000


def _valproj_body(src_ref, w_ref, b_ref, out_ref):
    r = (jnp.dot(src_ref[:], w_ref[:], preferred_element_type=jnp.float32)
         + b_ref[:])
    lo = jax.lax.bitcast_convert_type(
        r[:, :128].astype(jnp.bfloat16), jnp.uint16).astype(jnp.uint32)
    hi = jax.lax.bitcast_convert_type(
        r[:, 128:].astype(jnp.bfloat16), jnp.uint16).astype(jnp.uint32)
    out_ref[:] = jax.lax.bitcast_convert_type((hi << 16) | lo, jnp.float32)


def _valproj(src_f, W_val, b_val):
    rows = B * LIN
    grid = rows // _VBLK
    return pl.pallas_call(
        _valproj_body,
        grid=(grid,),
        in_specs=[
            pl.BlockSpec((_VBLK, D), lambda i: (i, 0)),
            pl.BlockSpec((D, D), lambda i: (0, 0)),
            pl.BlockSpec((1, D), lambda i: (0, 0)),
        ],
        out_specs=pl.BlockSpec((_VBLK, 128), lambda i: (i, 0)),
        out_shape=jax.ShapeDtypeStruct((rows, 128), jnp.float32),
    )(src_f, W_val, b_val)


_QBLK = 1024


def _prep_body(tgt_ref, rp_ref, woff_ref, boff_ref, wattn_ref, battn_ref,
               s_ref, a1_ref, myx_ref, w_out_ref, idx_out_ref):
    i = pl.program_id(0)
    b_f = (i // (LQ // _QBLK)).astype(jnp.float32)
    t = tgt_ref[:]
    off = jnp.dot(t, woff_ref[:], preferred_element_type=jnp.float32) + boff_ref[:]
    a = jnp.dot(t, wattn_ref[:], preferred_element_type=jnp.float32) + battn_ref[:]
    a = a - jnp.max(a, axis=1, keepdims=True)
    ea = jnp.exp(a)
    gs = jnp.dot(ea, s_ref[:], preferred_element_type=jnp.float32)
    aw = ea / gs

    lane64 = lax.broadcasted_iota(jnp.int32, (_QBLK, 64), 1)
    is_x = (lane64 % 2) == 0
    rp = rp_ref[:]
    refc = jnp.where(is_x, rp[:, 0:1], rp[:, 1:2])
    loc = (refc + off / 100.0) * 100.0 - 0.5
    fl = jnp.floor(loc)
    w1 = loc - fl
    w0 = 1.0 - w1
    v0 = ((fl >= 0.0) & (fl <= 99.0)).astype(jnp.float32)
    v1 = ((fl >= -1.0) & (fl <= 98.0)).astype(jnp.float32)
    c0 = jnp.clip(fl, 0.0, 99.0)
    c1 = jnp.clip(fl + 1.0, 0.0, 99.0)

    MYX = myx_ref[:]
    wyx = jnp.dot(jnp.concatenate([w0, w1], axis=1), MYX,
                  preferred_element_type=jnp.float32)
    vyx = jnp.dot(jnp.concatenate([v0, v1], axis=1), MYX,
                  preferred_element_type=jnp.float32)
    cyx = jnp.dot(jnp.concatenate([c0, c1], axis=1), MYX,
                  preferred_element_type=jnp.float32)
    aw128 = jnp.dot(aw, a1_ref[:], preferred_element_type=jnp.float32)

    h128 = (lax.broadcasted_iota(jnp.int32, (_QBLK, 128), 1) // 16).astype(jnp.float32)
    base = b_f * float(LIN * H)
    idxf = (cyx[:, :128] * 100.0 + cyx[:, 128:]) * 8.0 + h128 + base
    w_out_ref[:] = (aw128 * wyx[:, :128] * wyx[:, 128:]
                    * vyx[:, :128] * vyx[:, 128:])
    idx_out_ref[:] = idxf.astype(jnp.int32)


def _prep(tgt_f, rp_f, W_off, b_off, W_attn, b_attn):
    grid = (B * LQ) // _QBLK
    return pl.pallas_call(
        _prep_body,
        grid=(grid,),
        in_specs=[
            pl.BlockSpec((_QBLK, D), lambda i: (i, 0)),
            pl.BlockSpec((_QBLK, 2), lambda i: (i, 0)),
            pl.BlockSpec((D, 64), lambda i: (0, 0)),
            pl.BlockSpec((1, 64), lambda i: (0, 0)),
            pl.BlockSpec((D, 32), lambda i: (0, 0)),
            pl.BlockSpec((1, 32), lambda i: (0, 0)),
            pl.BlockSpec((32, 32), lambda i: (0, 0)),
            pl.BlockSpec((32, 128), lambda i: (0, 0)),
            pl.BlockSpec((128, 256), lambda i: (0, 0)),
        ],
        out_specs=[
            pl.BlockSpec((_QBLK, 128), lambda i: (i, 0)),
            pl.BlockSpec((_QBLK, 128), lambda i: (i, 0)),
        ],
        out_shape=[
            jax.ShapeDtypeStruct((B * LQ, 128), jnp.float32),
            jax.ShapeDtypeStruct((B * LQ, 128), jnp.int32),
        ],
    )(tgt_f, rp_f, W_off, b_off, W_attn, b_attn, _S, _A1, _MYX)


_NW = 32
_RPW = NOUT // _NW
_CH = 128
_NCHUNK = _RPW // _CH
_NGATH = (_CH * NPTS) // 128

_NBUF = 3


def _sc_gather_impl(table_hbm, idx_hbm, w_hbm, out_hbm, *scr):
    wid = lax.axis_index("s") * 2 + lax.axis_index("c")
    bufs = tuple(dict(idx=scr[b], w=scr[_NBUF + b], rows=scr[2 * _NBUF + b],
                      out=scr[3 * _NBUF + b], isem=scr[4 * _NBUF + b],
                      gsem=scr[5 * _NBUF + b], osem=scr[6 * _NBUF + b])
                 for b in range(_NBUF))

    def start_idxw(c, buf):
        base = wid * _RPW + c * _CH
        return [pltpu.async_copy(
                    idx_hbm.at[pl.ds(base * NPTS // 128, _NGATH)],
                    buf["idx"], buf["isem"]),
                pltpu.async_copy(
                    w_hbm.at[pl.ds(base * NPTS, _CH * NPTS)],
                    buf["w"], buf["isem"])]

    def fire(buf):
        return [pltpu.async_copy(table_hbm.at[buf["idx"].at[j]],
                                 buf["rows"].at[pl.ds(j * 128, 128)],
                                 buf["gsem"])
                for j in range(_NGATH)]

    def compute(c, buf, gathers):
        w_v, rows_v, out_v = buf["w"], buf["rows"], buf["out"]
        base = wid * _RPW + c * _CH
        for cp in gathers:
            cp.wait()

        @plsc.parallel_loop(0, _CH, 1, unroll=2)
        def j_body(j):
            wv = w_v[pl.ds(j * NPTS, 16)]
            acc0 = jnp.zeros((16,), jnp.float32)
            acc1 = jnp.zeros((16,), jnp.float32)
            for i in range(NPTS):
                k = j * NPTS + i
                wrd = rows_v[k, pl.ds(0, 16)]
                u = plsc.bitcast(wrd, jnp.uint32)
                g0 = plsc.bitcast(u << 16, jnp.float32)
                g1 = plsc.bitcast(u & jnp.uint32(0xFFFF0000), jnp.float32)
                ws = lax.gather(
                    wv, jnp.full((16, 1), i, jnp.int32),
                    lax.GatherDimensionNumbers(
                        offset_dims=(), collapsed_slice_dims=(0,),
                        start_index_map=(0,)),
                    (1,), mode=lax.GatherScatterMode.PROMISE_IN_BOUNDS)
                acc0 = acc0 + g0 * ws
                acc1 = acc1 + g1 * ws
            out_v[pl.ds(j * DH, 16)] = acc0
            out_v[pl.ds(j * DH + 16, 16)] = acc1
        return pltpu.async_copy(out_v, out_hbm.at[pl.ds(base * DH, _CH * DH)],
                                buf["osem"])

    iw = [None] * _NCHUNK
    gth = [None] * _NCHUNK
    ocp = [None] * _NBUF
    iw[0] = start_idxw(0, bufs[0])
    if _NCHUNK > 1:
        iw[1] = start_idxw(1, bufs[1])
    for cp in iw[0]:
        cp.wait()
    gth[0] = fire(bufs[0])
    for c in range(_NCHUNK):
        if c + 2 < _NCHUNK:
            iw[c + 2] = start_idxw(c + 2, bufs[(c + 2) % _NBUF])
        if c + 1 < _NCHUNK:
            for cp in iw[c + 1]:
                cp.wait()
            gth[c + 1] = fire(bufs[(c + 1) % _NBUF])
        if ocp[c % _NBUF] is not None:
            ocp[c % _NBUF].wait()
        ocp[c % _NBUF] = compute(c, bufs[c % _NBUF], gth[c])
    for oc in ocp:
        if oc is not None:
            oc.wait()


@functools.cache
def _sc_gather_kernel():
    mesh = plsc.VectorSubcoreMesh(
        core_axis_name="c", subcore_axis_name="s", num_cores=2, num_subcores=16)
    scratch = ([pltpu.VMEM((_NGATH, 128), jnp.int32)] * _NBUF
               + [pltpu.VMEM((_CH * NPTS,), jnp.float32)] * _NBUF
               + [pltpu.VMEM((_CH * NPTS, 16), jnp.float32)] * _NBUF
               + [pltpu.VMEM((_CH * DH,), jnp.float32)] * _NBUF
               + [pltpu.SemaphoreType.DMA] * (3 * _NBUF))
    return pl.kernel(
        _sc_gather_impl,
        out_type=jax.ShapeDtypeStruct((NOUT * DH,), jnp.float32),
        mesh=mesh,
        scratch_types=scratch,
        compiler_params=pltpu.CompilerParams(use_tc_tiling_on_sc=False,
                                             needs_layout_passes=False),
    )


_FBLK = 1024


def _final_body(attn_ref, tgt_ref, w_ref, b_ref, g_ref, bn_ref, out_ref):
    y = (jnp.dot(attn_ref[:], w_ref[:], preferred_element_type=jnp.float32)
         + b_ref[:] + tgt_ref[:])
    m = jnp.mean(y, axis=1, keepdims=True)
    yc = y - m
    v = jnp.mean(yc * yc, axis=1, keepdims=True)
    out_ref[:] = yc / jnp.sqrt(v + 1e-5) * g_ref[:] + bn_ref[:]


def _final(attn_f, tgt_f, W_out, b_out, g1, b1n):
    grid = (B * LQ) // _FBLK
    return pl.pallas_call(
        _final_body,
        grid=(grid,),
        in_specs=[
            pl.BlockSpec((_FBLK, D), lambda i: (i, 0)),
            pl.BlockSpec((_FBLK, D), lambda i: (i, 0)),
            pl.BlockSpec((D, D), lambda i: (0, 0)),
            pl.BlockSpec((1, D), lambda i: (0, 0)),
            pl.BlockSpec((1, D), lambda i: (0, 0)),
            pl.BlockSpec((1, D), lambda i: (0, 0)),
        ],
        out_specs=pl.BlockSpec((_FBLK, D), lambda i: (i, 0)),
        out_shape=jax.ShapeDtypeStruct((B * LQ, D), jnp.float32),
    )(attn_f, tgt_f, W_out, b_out, g1, b1n)


def kernel(tgt, src, reference_points, spatial_shapes, level_start_index,
           W_off, b_off, W_attn, b_attn, W_val, b_val, W_out, b_out,
           g1, b1n, W1, bl1, W2, bl2, g2, b2n):
    tgt_f = tgt.reshape(B * LQ, D)
    src_f = src.reshape(B * LIN, D)
    rp_f = reference_points.reshape(B * LQ, 2)

    Wv3 = W_val.reshape(D, H, DH)
    Wp = jnp.concatenate([Wv3[:, :, :16].reshape(D, H * 16),
                          Wv3[:, :, 16:].reshape(D, H * 16)], axis=1)
    bv = b_val.reshape(H, DH)
    bp = jnp.concatenate([bv[:, :16].reshape(-1), bv[:, 16:].reshape(-1)])
    value = _valproj(src_f, Wp, bp.reshape(1, D))
    w128, idx128 = _prep(tgt_f, rp_f, W_off, b_off.reshape(1, 64),
                         W_attn, b_attn.reshape(1, 32))

    attn_flat = _sc_gather_kernel()(value.reshape(B * LIN * H, 16), idx128,
                                    w128.reshape(-1))

    out = _final(attn_flat.reshape(B * LQ, D), tgt_f,
                 W_out, b_out.reshape(1, D), g1.reshape(1, D), b1n.reshape(1, D))
    return out.reshape(B, LQ, D)


# speedup vs baseline: 1.0965x; 1.0104x over previous
import functools

import jax
import jax.numpy as jnp
import numpy as np
from jax import lax
from jax.experimental import pallas as pl
from jax.experimental.pallas import tpu as pltpu
from jax.experimental.pallas import tpu_sc as plsc

B, LQ, D = 4, 1024, 256
H, P = 8, 4
HH, WW = 100, 100
LIN = HH * WW
DH = D // H
NPTS = P * 4
NOUT = B * LQ * H

_S = np.zeros((32, 32), np.float32)
for _i in range(32):
    for _j in range(32):
        if _i // 4 == _j // 4:
            _S[_i, _j] = 1.0
_A1 = np.zeros((32, 128), np.float32)
_MYX = np.zeros((128, 256), np.float32)
for _h in range(H):
    for _p in range(P):
        for _c in range(4):
            _l = _h * 16 + _p * 4 + _c
            _A1[_h * 4 + _p, _l] = 1.0
            _MYX[(_c >> 1) * 64 + _h * 8 + _p * 2 + 1, _l] = 1.0
            _MYX[(_c & 1) * 64 + _h * 8 + _p * 2 + 0, 128 + _l] = 1.0

_VBLK = 8000


def _valproj_body(src_ref, w_ref, b_ref, out_ref):
    r = (jnp.dot(src_ref[:], w_ref[:], preferred_element_type=jnp.float32)
         + b_ref[:])
    lo = jax.lax.bitcast_convert_type(
        r[:, :128].astype(jnp.bfloat16), jnp.uint16).astype(jnp.uint32)
    hi = jax.lax.bitcast_convert_type(
        r[:, 128:].astype(jnp.bfloat16), jnp.uint16).astype(jnp.uint32)
    out_ref[:] = jax.lax.bitcast_convert_type((hi << 16) | lo, jnp.float32)


def _valproj(src_f, W_val, b_val):
    rows = B * LIN
    grid = rows // _VBLK
    return pl.pallas_call(
        _valproj_body,
        grid=(grid,),
        in_specs=[
            pl.BlockSpec((_VBLK, D), lambda i: (i, 0)),
            pl.BlockSpec((D, D), lambda i: (0, 0)),
            pl.BlockSpec((1, D), lambda i: (0, 0)),
        ],
        out_specs=pl.BlockSpec((_VBLK, 128), lambda i: (i, 0)),
        out_shape=jax.ShapeDtypeStruct((rows, 128), jnp.float32),
    )(src_f, W_val, b_val)


_QBLK = 1024


def _prep_body(tgt_ref, rp_ref, woff_ref, boff_ref, wattn_ref, battn_ref,
               s_ref, a1_ref, myx_ref, w_out_ref, idx_out_ref):
    i = pl.program_id(0)
    b_f = (i // (LQ // _QBLK)).astype(jnp.float32)
    t = tgt_ref[:]
    off = jnp.dot(t, woff_ref[:], preferred_element_type=jnp.float32) + boff_ref[:]
    a = jnp.dot(t, wattn_ref[:], preferred_element_type=jnp.float32) + battn_ref[:]
    a = a - jnp.max(a, axis=1, keepdims=True)
    ea = jnp.exp(a)
    gs = jnp.dot(ea, s_ref[:], preferred_element_type=jnp.float32)
    aw = ea / gs

    lane64 = lax.broadcasted_iota(jnp.int32, (_QBLK, 64), 1)
    is_x = (lane64 % 2) == 0
    rp = rp_ref[:]
    refc = jnp.where(is_x, rp[:, 0:1], rp[:, 1:2])
    loc = (refc + off / 100.0) * 100.0 - 0.5
    fl = jnp.floor(loc)
    w1 = loc - fl
    w0 = 1.0 - w1
    v0 = ((fl >= 0.0) & (fl <= 99.0)).astype(jnp.float32)
    v1 = ((fl >= -1.0) & (fl <= 98.0)).astype(jnp.float32)
    c0 = jnp.clip(fl, 0.0, 99.0)
    c1 = jnp.clip(fl + 1.0, 0.0, 99.0)

    MYX = myx_ref[:]
    wyx = jnp.dot(jnp.concatenate([w0, w1], axis=1), MYX,
                  preferred_element_type=jnp.float32)
    vyx = jnp.dot(jnp.concatenate([v0, v1], axis=1), MYX,
                  preferred_element_type=jnp.float32)
    cyx = jnp.dot(jnp.concatenate([c0, c1], axis=1), MYX,
                  preferred_element_type=jnp.float32)
    aw128 = jnp.dot(aw, a1_ref[:], preferred_element_type=jnp.float32)

    h128 = (lax.broadcasted_iota(jnp.int32, (_QBLK, 128), 1) // 16).astype(jnp.float32)
    base = b_f * float(LIN * H)
    idxf = (cyx[:, :128] * 100.0 + cyx[:, 128:]) * 8.0 + h128 + base
    w_out_ref[:] = (aw128 * wyx[:, :128] * wyx[:, 128:]
                    * vyx[:, :128] * vyx[:, 128:])
    idx_out_ref[:] = idxf.astype(jnp.int32)


def _prep(tgt_f, rp_f, W_off, b_off, W_attn, b_attn):
    grid = (B * LQ) // _QBLK
    return pl.pallas_call(
        _prep_body,
        grid=(grid,),
        in_specs=[
            pl.BlockSpec((_QBLK, D), lambda i: (i, 0)),
            pl.BlockSpec((_QBLK, 2), lambda i: (i, 0)),
            pl.BlockSpec((D, 64), lambda i: (0, 0)),
            pl.BlockSpec((1, 64), lambda i: (0, 0)),
            pl.BlockSpec((D, 32), lambda i: (0, 0)),
            pl.BlockSpec((1, 32), lambda i: (0, 0)),
            pl.BlockSpec((32, 32), lambda i: (0, 0)),
            pl.BlockSpec((32, 128), lambda i: (0, 0)),
            pl.BlockSpec((128, 256), lambda i: (0, 0)),
        ],
        out_specs=[
            pl.BlockSpec((_QBLK, 128), lambda i: (i, 0)),
            pl.BlockSpec((_QBLK, 128), lambda i: (i, 0)),
        ],
        out_shape=[
            jax.ShapeDtypeStruct((B * LQ, 128), jnp.float32),
            jax.ShapeDtypeStruct((B * LQ, 128), jnp.int32),
        ],
    )(tgt_f, rp_f, W_off, b_off, W_attn, b_attn, _S, _A1, _MYX)


_NW = 32
_RPW = NOUT // _NW
_CH = 128
_NCHUNK = _RPW // _CH
_NGATH = (_CH * NPTS) // 128

_NBUF = 3


def _sc_gather_impl(table_hbm, idx_hbm, w_hbm, out_hbm, *scr):
    wid = lax.axis_index("s") * 2 + lax.axis_index("c")
    bufs = tuple(dict(idx=scr[b], w=scr[_NBUF + b], rows=scr[2 * _NBUF + b],
                      out=scr[3 * _NBUF + b], isem=scr[4 * _NBUF + b],
                      gsem=scr[5 * _NBUF + b], osem=scr[6 * _NBUF + b])
                 for b in range(_NBUF))

    def start_idxw(c, buf):
        base = wid * _RPW + c * _CH
        return [pltpu.async_copy(
                    idx_hbm.at[pl.ds(base * NPTS // 128, _NGATH)],
                    buf["idx"], buf["isem"]),
                pltpu.async_copy(
                    w_hbm.at[pl.ds(base * NPTS, _CH * NPTS)],
                    buf["w"], buf["isem"])]

    def fire(buf):
        return [pltpu.async_copy(table_hbm.at[buf["idx"].at[j]],
                                 buf["rows"].at[pl.ds(j * 128, 128)],
                                 buf["gsem"])
                for j in range(_NGATH)]

    def compute(c, buf, gathers):
        w_v, rows_v, out_v = buf["w"], buf["rows"], buf["out"]
        base = wid * _RPW + c * _CH
        for cp in gathers:
            cp.wait()

        @plsc.parallel_loop(0, _CH, 1, unroll=2)
        def j_body(j):
            wv = w_v[pl.ds(j * NPTS, 16)]
            acc0 = jnp.zeros((16,), jnp.float32)
            acc1 = jnp.zeros((16,), jnp.float32)
            for i in range(NPTS):
                k = j * NPTS + i
                wrd = rows_v[k, pl.ds(0, 16)]
                u = plsc.bitcast(wrd, jnp.uint32)
                g0 = plsc.bitcast(u << 16, jnp.float32)
                g1 = plsc.bitcast(u & jnp.uint32(0xFFFF0000), jnp.float32)
                ws = lax.gather(
                    wv, jnp.full((16, 1), i, jnp.int32),
                    lax.GatherDimensionNumbers(
                        offset_dims=(), collapsed_slice_dims=(0,),
                        start_index_map=(0,)),
                    (1,), mode=lax.GatherScatterMode.PROMISE_IN_BOUNDS)
                acc0 = acc0 + g0 * ws
                acc1 = acc1 + g1 * ws
            out_v[pl.ds(j * DH, 16)] = acc0
            out_v[pl.ds(j * DH + 16, 16)] = acc1
        return pltpu.async_copy(out_v, out_hbm.at[pl.ds(base * DH, _CH * DH)],
                                buf["osem"])

    iw = [None] * _NCHUNK
    gth = [None] * _NCHUNK
    ocp = [None] * _NBUF
    iw[0] = start_idxw(0, bufs[0])
    if _NCHUNK > 1:
        iw[1] = start_idxw(1, bufs[1])
    for cp in iw[0]:
        cp.wait()
    gth[0] = fire(bufs[0])
    for c in range(_NCHUNK):
        if c + 2 < _NCHUNK:
            iw[c + 2] = start_idxw(c + 2, bufs[(c + 2) % _NBUF])
        if c + 1 < _NCHUNK:
            for cp in iw[c + 1]:
                cp.wait()
            gth[c + 1] = fire(bufs[(c + 1) % _NBUF])
        if ocp[c % _NBUF] is not None:
            ocp[c % _NBUF].wait()
        ocp[c % _NBUF] = compute(c, bufs[c % _NBUF], gth[c])
    for oc in ocp:
        if oc is not None:
            oc.wait()


@functools.cache
def _sc_gather_kernel():
    mesh = plsc.VectorSubcoreMesh(
        core_axis_name="c", subcore_axis_name="s", num_cores=2, num_subcores=16)
    scratch = ([pltpu.VMEM((_NGATH, 128), jnp.int32)] * _NBUF
               + [pltpu.VMEM((_CH * NPTS,), jnp.float32)] * _NBUF
               + [pltpu.VMEM((_CH * NPTS, 16), jnp.float32)] * _NBUF
               + [pltpu.VMEM((_CH * DH,), jnp.float32)] * _NBUF
               + [pltpu.SemaphoreType.DMA] * (3 * _NBUF))
    return pl.kernel(
        _sc_gather_impl,
        out_type=jax.ShapeDtypeStruct((NOUT * DH,), jnp.float32),
        mesh=mesh,
        scratch_types=scratch,
        compiler_params=pltpu.CompilerParams(use_tc_tiling_on_sc=False,
                                             needs_layout_passes=False),
    )


_FBLK = 1024


def _final_body(attn_ref, tgt_ref, w_ref, b_ref, g_ref, bn_ref, out_ref):
    y = (jnp.dot(attn_ref[:], w_ref[:], preferred_element_type=jnp.float32)
         + b_ref[:] + tgt_ref[:])
    m = jnp.mean(y, axis=1, keepdims=True)
    yc = y - m
    v = jnp.mean(yc * yc, axis=1, keepdims=True)
    out_ref[:] = yc / jnp.sqrt(v + 1e-5) * g_ref[:] + bn_ref[:]


def _final(attn_f, tgt_f, W_out, b_out, g1, b1n):
    grid = (B * LQ) // _FBLK
    return pl.pallas_call(
        _final_body,
        grid=(grid,),
        in_specs=[
            pl.BlockSpec((_FBLK, D), lambda i: (i, 0)),
            pl.BlockSpec((_FBLK, D), lambda i: (i, 0)),
            pl.BlockSpec((D, D), lambda i: (0, 0)),
            pl.BlockSpec((1, D), lambda i: (0, 0)),
            pl.BlockSpec((1, D), lambda i: (0, 0)),
            pl.BlockSpec((1, D), lambda i: (0, 0)),
        ],
        out_specs=pl.BlockSpec((_FBLK, D), lambda i: (i, 0)),
        out_shape=jax.ShapeDtypeStruct((B * LQ, D), jnp.float32),
    )(attn_f, tgt_f, W_out, b_out, g1, b1n)


def kernel(tgt, src, reference_points, spatial_shapes, level_start_index,
           W_off, b_off, W_attn, b_attn, W_val, b_val, W_out, b_out,
           g1, b1n, W1, bl1, W2, bl2, g2, b2n):
    tgt_f = tgt.reshape(B * LQ, D)
    src_f = src.reshape(B * LIN, D)
    rp_f = reference_points.reshape(B * LQ, 2)

    Wv3 = W_val.reshape(D, H, DH)
    Wp = jnp.concatenate([Wv3[:, :, :16].reshape(D, H * 16),
                          Wv3[:, :, 16:].reshape(D, H * 16)], axis=1)
    bv = b_val.reshape(H, DH)
    bp = jnp.concatenate([bv[:, :16].reshape(-1), bv[:, 16:].reshape(-1)])
    value = _valproj(src_f, Wp, bp.reshape(1, D))
    w128, idx128 = _prep(tgt_f, rp_f, W_off, b_off.reshape(1, 64),
                         W_attn, b_attn.reshape(1, 32))

    attn_flat = _sc_gather_kernel()(value.reshape(B * LIN * H, 16), idx128,
                                    w128.reshape(-1))

    out = _final(attn_flat.reshape(B * LQ, D), tgt_f,
                 W_out, b_out.reshape(1, D), g1.reshape(1, D), b1n.reshape(1, D))
    return out.reshape(B, LQ, D)
